# Initial kernel scaffold; baseline (speedup 1.0000x reference)
import jax
import jax.numpy as jnp
from jax.experimental import pallas as pl


def kernel(x, edge_index, edge_attr, batch, W1, W2, Wend):
    raise NotImplementedError("write your pallas kernel here")


# speedup vs baseline: 8.3360x; 8.3360x over previous
import functools

import jax
import jax.numpy as jnp
from jax.experimental import pallas as pl
from jax.experimental.pallas import tpu as pltpu

N_NODES = 10000
N_GRAPHS = 64
BLOCK = 2000
NUM_BLOCKS = N_NODES // BLOCK


def _fused_body(x_ref, batch_ref, w1_ref, w2_ref, wend_ref, out_ref,
                sums_ref, counts_ref):
    i = pl.program_id(0)

    @pl.when(i == 0)
    def _init():
        sums_ref[...] = jnp.zeros_like(sums_ref)
        counts_ref[...] = jnp.zeros_like(counts_ref)

    x = x_ref[...]
    h = jax.lax.dot(x, w1_ref[...],
                    preferred_element_type=jnp.float32)
    h = jnp.maximum(h, 0.0)
    h = jax.lax.dot(h, w2_ref[...],
                    preferred_element_type=jnp.float32)
    h = jnp.maximum(h, 0.0)
    h3 = jax.lax.dot(h, wend_ref[...],
                     preferred_element_type=jnp.float32)

    b = batch_ref[0]
    seg = jax.lax.broadcasted_iota(jnp.int32, (N_GRAPHS, BLOCK), 0)
    maskf = (b == seg).astype(jnp.float32)
    sums_ref[...] += jax.lax.dot(maskf, h3,
                                 preferred_element_type=jnp.float32)
    counts_ref[...] += jnp.sum(maskf, axis=1, keepdims=True)

    @pl.when(i == NUM_BLOCKS - 1)
    def _fin():
        pooled = sums_ref[...] / jnp.maximum(counts_ref[...], 1.0)
        out_ref[...] = jax.nn.sigmoid(pooled)


@functools.partial(jax.jit, static_argnames=())
def _fused_call(x, batch3, W1, W2, Wend):
    return pl.pallas_call(
        _fused_body,
        grid=(NUM_BLOCKS,),
        in_specs=[
            pl.BlockSpec((BLOCK, 128), lambda i: (i, 0)),
            pl.BlockSpec((1, 1, BLOCK), lambda i: (i, 0, 0)),
            pl.BlockSpec((128, 128), lambda i: (0, 0)),
            pl.BlockSpec((128, 128), lambda i: (0, 0)),
            pl.BlockSpec((128, 1), lambda i: (0, 0)),
        ],
        out_specs=pl.BlockSpec((N_GRAPHS, 1), lambda i: (0, 0)),
        out_shape=jax.ShapeDtypeStruct((N_GRAPHS, 1), jnp.float32),
        scratch_shapes=[
            pltpu.VMEM((N_GRAPHS, 1), jnp.float32),
            pltpu.VMEM((N_GRAPHS, 1), jnp.float32),
        ],
        compiler_params=pltpu.CompilerParams(
            dimension_semantics=("arbitrary",),
        ),
    )(x, batch3, W1, W2, Wend)


def kernel(x, edge_index, edge_attr, batch, W1, W2, Wend):
    del edge_index, edge_attr
    batch3 = batch.reshape(NUM_BLOCKS, 1, BLOCK)
    return _fused_call(x, batch3, W1, W2, Wend)
